# baseline (speedup 1.0000x reference)
import jax
import jax.numpy as jnp
from jax import lax
from jax.experimental import pallas as pl
from jax.experimental.pallas import tpu as pltpu
from jax.experimental.pallas import tpu_sc as plsc

D_MODEL = 64
B = 4096
L = 200

_info = plsc.get_sparse_core_info()
NC = _info.num_cores
NS = _info.num_subcores
NW = NC * NS

ROWS_PER_W = B // NW
G = ROWS_PER_W // 2


def _sc_body(px_hbm, py_hbm, ex_hbm, ey_hbm, out_hbm,
             ix0, iy0, ix1, iy1, r0, r1, sa0, sb0, sc0, sa1, sb1, sc1):
  wid = lax.axis_index("s") * NC + lax.axis_index("c")
  row0 = wid * ROWS_PER_W

  def stage_idx(b, ix, iy):
    pltpu.sync_copy(px_hbm.at[b], ix)
    pltpu.sync_copy(py_hbm.at[b], iy)

  stage_idx(row0, ix0, iy0)
  pltpu.async_copy(ex_hbm.at[ix0], r0, sa0)

  def g_body(g, _):
    b0 = row0 + 2 * g
    b1 = b0 + 1

    pltpu.make_async_copy(ex_hbm.at[ix0], r0, sa0).wait()
    cpb0 = pltpu.async_copy(ey_hbm.at[iy0], r0, sb0, add=True)

    @pl.when(g >= 1)
    def _():
      pltpu.make_async_copy(r1, out_hbm.at[b0 - 1], sc1).wait()

    stage_idx(b1, ix1, iy1)
    pltpu.async_copy(ex_hbm.at[ix1], r1, sa1)
    cpb0.wait()
    pltpu.async_copy(r0, out_hbm.at[b0], sc0)

    pltpu.make_async_copy(ex_hbm.at[ix1], r1, sa1).wait()
    cpb1 = pltpu.async_copy(ey_hbm.at[iy1], r1, sb1, add=True)

    @pl.when(g + 1 < G)
    def _():
      pltpu.make_async_copy(r0, out_hbm.at[b0], sc0).wait()
      stage_idx(b0 + 2, ix0, iy0)
      pltpu.async_copy(ex_hbm.at[ix0], r0, sa0)

    cpb1.wait()
    pltpu.async_copy(r1, out_hbm.at[b1], sc1)
    return 0

  lax.fori_loop(0, G, g_body, 0)

  pltpu.make_async_copy(r0, out_hbm.at[row0 + ROWS_PER_W - 2], sc0).wait()
  pltpu.make_async_copy(r1, out_hbm.at[row0 + ROWS_PER_W - 1], sc1).wait()


_mesh = plsc.VectorSubcoreMesh(core_axis_name="c", subcore_axis_name="s")

_sc_kernel = pl.kernel(
    _sc_body,
    out_type=jax.ShapeDtypeStruct((B, L, D_MODEL), jnp.float32),
    mesh=_mesh,
    scratch_types=[
        pltpu.VMEM((L,), jnp.int32),
        pltpu.VMEM((L,), jnp.int32),
        pltpu.VMEM((L,), jnp.int32),
        pltpu.VMEM((L,), jnp.int32),
        pltpu.VMEM((L, D_MODEL), jnp.float32),
        pltpu.VMEM((L, D_MODEL), jnp.float32),
        pltpu.SemaphoreType.DMA,
        pltpu.SemaphoreType.DMA,
        pltpu.SemaphoreType.DMA,
        pltpu.SemaphoreType.DMA,
        pltpu.SemaphoreType.DMA,
        pltpu.SemaphoreType.DMA,
    ],
    compiler_params=pltpu.CompilerParams(use_tc_tiling_on_sc=False),
)


@jax.jit
def kernel(pos_x, pos_y, ex_weight, ey_weight):
  px = pos_x.astype(jnp.int32)
  py = pos_y.astype(jnp.int32)
  return _sc_kernel(px, py, ex_weight, ey_weight)


# speedup vs baseline: 1.3660x; 1.3660x over previous
import jax
import jax.numpy as jnp
from jax import lax
from jax.experimental import pallas as pl
from jax.experimental.pallas import tpu as pltpu
from jax.experimental.pallas import tpu_sc as plsc

D_MODEL = 64
B = 4096
L = 200
N = B * L
VOCAB = 1001

_info = plsc.get_sparse_core_info()
NC = _info.num_cores
NS = _info.num_subcores
NW = NC * NS

CHUNK = 512
PER_W = N // NW
N_CHUNKS = PER_W // CHUNK
G = N_CHUNKS // 2


def _sc_body(px_hbm, py_hbm, ex_hbm, ey_hbm, out_hbm,
             sh_ex, sh_ey, ix0, iy0, ix1, iy1, r0, r1,
             sa0, sb0, sc0, sa1, sb1, sc1):
  wid = lax.axis_index("s") * NC + lax.axis_index("c")
  w_base = wid * PER_W

  @pl.when(lax.axis_index("s") == 0)
  def _():
    pltpu.sync_copy(ex_hbm, sh_ex)
    pltpu.sync_copy(ey_hbm, sh_ey)

  plsc.subcore_barrier()

  def stage_idx(c, ix, iy):
    base = w_base + c * CHUNK
    pltpu.sync_copy(px_hbm.at[pl.ds(base, CHUNK)], ix)
    pltpu.sync_copy(py_hbm.at[pl.ds(base, CHUNK)], iy)

  def out_slice(c):
    return out_hbm.at[pl.ds(w_base + c * CHUNK, CHUNK)]

  stage_idx(0, ix0, iy0)
  pltpu.async_copy(sh_ex.at[ix0], r0, sa0)

  def g_body(g, _):
    c0 = 2 * g
    c1 = c0 + 1

    pltpu.make_async_copy(sh_ex.at[ix0], r0, sa0).wait()
    cpb0 = pltpu.async_copy(sh_ey.at[iy0], r0, sb0, add=True)

    @pl.when(g >= 1)
    def _():
      pltpu.make_async_copy(r1, out_slice(c0 - 1), sc1).wait()

    stage_idx(c1, ix1, iy1)
    pltpu.async_copy(sh_ex.at[ix1], r1, sa1)
    cpb0.wait()
    pltpu.async_copy(r0, out_slice(c0), sc0)

    pltpu.make_async_copy(sh_ex.at[ix1], r1, sa1).wait()
    cpb1 = pltpu.async_copy(sh_ey.at[iy1], r1, sb1, add=True)

    @pl.when(g + 1 < G)
    def _():
      pltpu.make_async_copy(r0, out_slice(c0), sc0).wait()
      stage_idx(c0 + 2, ix0, iy0)
      pltpu.async_copy(sh_ex.at[ix0], r0, sa0)

    cpb1.wait()
    pltpu.async_copy(r1, out_slice(c1), sc1)
    return 0

  lax.fori_loop(0, G, g_body, 0)

  pltpu.make_async_copy(r0, out_slice(N_CHUNKS - 2), sc0).wait()
  pltpu.make_async_copy(r1, out_slice(N_CHUNKS - 1), sc1).wait()


_mesh = plsc.VectorSubcoreMesh(core_axis_name="c", subcore_axis_name="s")

_sc_kernel = pl.kernel(
    _sc_body,
    out_type=jax.ShapeDtypeStruct((N, D_MODEL), jnp.float32),
    mesh=_mesh,
    scratch_types=[
        pltpu.VMEM_SHARED((VOCAB, D_MODEL), jnp.float32),
        pltpu.VMEM_SHARED((VOCAB, D_MODEL), jnp.float32),
        pltpu.VMEM((CHUNK,), jnp.int32),
        pltpu.VMEM((CHUNK,), jnp.int32),
        pltpu.VMEM((CHUNK,), jnp.int32),
        pltpu.VMEM((CHUNK,), jnp.int32),
        pltpu.VMEM((CHUNK, D_MODEL), jnp.float32),
        pltpu.VMEM((CHUNK, D_MODEL), jnp.float32),
        pltpu.SemaphoreType.DMA,
        pltpu.SemaphoreType.DMA,
        pltpu.SemaphoreType.DMA,
        pltpu.SemaphoreType.DMA,
        pltpu.SemaphoreType.DMA,
        pltpu.SemaphoreType.DMA,
    ],
    compiler_params=pltpu.CompilerParams(use_tc_tiling_on_sc=False),
)


@jax.jit
def kernel(pos_x, pos_y, ex_weight, ey_weight):
  px = pos_x.reshape(N).astype(jnp.int32)
  py = pos_y.reshape(N).astype(jnp.int32)
  out = _sc_kernel(px, py, ex_weight, ey_weight)
  return out.reshape(B, L, D_MODEL)
